# baseline (speedup 1.0000x reference)
import functools

import jax
import jax.numpy as jnp
from jax import lax
from jax.experimental import pallas as pl
from jax.experimental.pallas import tpu as pltpu
from jax.experimental.pallas import tpu_sc as plsc

N = 10000
E = 160000
F = 256
FH = 128
EPT = E // 16
CHUNK = 80
NCH = EPT // CHUNK
NP = 10240
NPT = NP // 16
ESLAB = 40
MROWS = 1000


def _mm_body(x_ref, w_ref, b_ref, o_ref):
    o_ref[0] = (lax.dot_general(
        x_ref[...], w_ref[...], (((1,), (1,)), ((), ())),
        preferred_element_type=jnp.float32)
        + b_ref[pl.ds(pl.program_id(0), 1)]).astype(jnp.bfloat16)


def _linear(x, W, b):
    return pl.pallas_call(
        _mm_body,
        grid=(2, N // MROWS),
        in_specs=[
            pl.BlockSpec((MROWS, F), lambda c, i: (i, 0)),
            pl.BlockSpec((FH, F), lambda c, i: (c, 0)),
            pl.BlockSpec((2, FH), lambda c, i: (0, 0)),
        ],
        out_specs=pl.BlockSpec((1, MROWS, FH), lambda c, i: (c, i, 0)),
        out_shape=jax.ShapeDtypeStruct((2, N, FH), jnp.bfloat16),
    )(x, W, b.reshape(2, FH))


def _make_spmm(table_rows_per_core: int, scale_w: bool, out_bf16: bool):
    TN = table_rows_per_core
    mesh = plsc.VectorSubcoreMesh(core_axis_name="c", subcore_axis_name="s")
    out_dtype = jnp.bfloat16 if out_bf16 else jnp.float32

    @functools.partial(
        pl.kernel, mesh=mesh,
        out_type=jax.ShapeDtypeStruct((2, NP, FH), out_dtype),
        compiler_params=pltpu.CompilerParams(
            use_tc_tiling_on_sc=False, needs_layout_passes=False),
        scratch_types=[
            pltpu.VMEM_SHARED((NP, FH), jnp.float32),
            pltpu.VMEM((2, CHUNK, FH), jnp.bfloat16),
            pltpu.VMEM((2, CHUNK, FH), jnp.float32),
            pltpu.VMEM((2, ESLAB, FH), jnp.float32),
            pltpu.VMEM((2, ESLAB, FH), jnp.bfloat16),
            pltpu.VMEM((2, CHUNK), jnp.int32),
            pltpu.VMEM((4, CHUNK), jnp.int32),
            pltpu.VMEM((4, CHUNK), jnp.int32),
            pltpu.VMEM((4, CHUNK), jnp.float32),
            pltpu.VMEM((16,), jnp.float32),
            pltpu.SemaphoreType.DMA,
            pltpu.SemaphoreType.DMA,
            pltpu.SemaphoreType.DMA,
        ],
    )
    def spmm(table, col, dst, ew, scal, out,
             acc, rowsv, srows, ebuf, obuf, idxv, colv, dstv, wv, scalv,
             sem, msem, ssem):
        c = lax.axis_index("c")
        s = lax.axis_index("s")
        zero16 = jnp.zeros((16,), jnp.float32)
        iota = lax.iota(jnp.int32, 16)
        eidx = [iota * 2 + 32 * j for j in range(4)]
        oidx = [iota * 2 + 1 + 32 * j for j in range(4)]
        pltpu.sync_copy(scal, scalv)
        sv = scalv[pl.ds(0, 16)]

        def zb(e, cc):
            for f in range(FH // 16):
                srows[0, e, pl.ds(f * 16, 16)] = zero16
            return cc
        lax.fori_loop(0, CHUNK, zb, 0)
        for j in range(NPT // CHUNK):
            pltpu.sync_copy(srows.at[0],
                            acc.at[pl.ds(s * NPT + j * CHUNK, CHUNK)])
        plsc.subcore_barrier()

        cN = c * TN
        cNv = jnp.full((16,), cN, jnp.int32)
        ebase = s * EPT

        def stage_meta(e0, q):
            cp1 = pltpu.async_copy(col.at[pl.ds(e0, CHUNK)], colv.at[q], msem)
            cp2 = pltpu.async_copy(dst.at[pl.ds(e0, CHUNK)], dstv.at[q], msem)
            cp3 = pltpu.async_copy(ew.at[pl.ds(e0, CHUNK)], wv.at[q], msem)
            return cp1, cp2, cp3

        def build_idx(pi, q):
            def ib(i, c2):
                idxv[pi, pl.ds(i * 16, 16)] = colv[q, pl.ds(i * 16, 16)] + cNv
                return c2
            lax.fori_loop(0, CHUNK // 16, ib, 0)
            if scale_w:
                def wb(i, c2):
                    wv[q, pl.ds(i * 16, 16)] = wv[q, pl.ds(i * 16, 16)] * sv
                    return c2
                lax.fori_loop(0, CHUNK // 16, wb, 0)

        def fire_gather(p):
            return pltpu.async_copy(table.at[idxv.at[p]], rowsv.at[p], sem)

        def wait_gather(p):
            pltpu.make_async_copy(table.at[idxv.at[p]], rowsv.at[p], sem).wait()

        pconst = [jnp.full((16,), pp, jnp.int32) for pp in range(4)]

        def scale(p, q):
            @plsc.parallel_loop(0, CHUNK, 1, unroll=2)
            def sb(e):
                ws = plsc.load_gather(
                    wv, [pconst[q], jnp.full((16,), e, jnp.int32)])
                ev = jnp.full((16,), e, jnp.int32)
                for j in range(4):
                    u = rowsv[p, e, pl.ds(j * 32, 32)]
                    a, b = plsc.unpack(u, format=plsc.PackFormat.INTERLEAVED)
                    plsc.store_scatter(srows, [pconst[p], ev, eidx[j]], a * ws)
                    plsc.store_scatter(srows, [pconst[p], ev, oidx[j]], b * ws)

        def fire_scatter(p, q):
            return pltpu.async_copy(srows.at[p], acc.at[dstv.at[q]],
                                    ssem, add=True)

        def wait_scatter(p, q):
            pltpu.make_async_copy(srows.at[p], acc.at[dstv.at[q]],
                                  ssem).wait()

        emax = E - CHUNK

        def wait_meta(q):
            for r in (colv, dstv):
                pltpu.make_async_copy(col.at[pl.ds(0, CHUNK)],
                                      r.at[q], msem).wait()
            pltpu.make_async_copy(ew.at[pl.ds(0, CHUNK)],
                                  wv.at[q], msem).wait()

        for cp in stage_meta(ebase, 0):
            cp.wait()
        build_idx(0, 0)
        fire_gather(0)
        stage_meta(ebase + CHUNK, 1)

        def half(i, g, j):
            p = j % 2
            wait_meta((j + 1) % 4)
            build_idx(1 - p, (j + 1) % 4)
            wait_gather(p)
            if j == 0:
                @pl.when(i > 0)
                def _():
                    wait_scatter(1, 3)
            else:
                wait_scatter(1 - p, j - 1)
            fire_gather(1 - p)
            scale(p, j)
            fire_scatter(p, j)
            stage_meta(jnp.minimum(ebase + (g + 2) * CHUNK, emax),
                       (j + 2) % 4)

        def quad_body(i, cc):
            g = 4 * i
            for j in range(4):
                half(i, g + j, j)
            return cc
        lax.fori_loop(0, (NCH - 1) // 4, quad_body, 0)

        wait_meta(1)
        wait_gather(0)
        wait_scatter(1, 3)
        scale(0, 0)
        pltpu.sync_copy(srows.at[0], acc.at[dstv.at[0]], add=True)

        plsc.subcore_barrier()
        nslab = NPT // ESLAB
        rbase = s * NPT
        pltpu.async_copy(acc.at[pl.ds(rbase, ESLAB)], ebuf.at[0], msem)
        outcps = []
        for k in range(nslab):
            ep = k % 2
            pltpu.make_async_copy(acc.at[pl.ds(rbase, ESLAB)],
                                  ebuf.at[ep], msem).wait()
            if k + 1 < nslab:
                if not out_bf16 and k + 1 >= 2:
                    outcps[k - 1].wait()
                pltpu.async_copy(
                    acc.at[pl.ds(rbase + (k + 1) * ESLAB, ESLAB)],
                    ebuf.at[1 - ep], msem)

            def eb(r, cc, ep=ep):
                for f in range(FH // 16):
                    v = ebuf[ep, r, pl.ds(f * 16, 16)]
                    ebuf[ep, r, pl.ds(f * 16, 16)] = jnp.where(
                        v > 0, v, jnp.exp(v) - 1.0)
                return cc
            lax.fori_loop(0, ESLAB, eb, 0)

            if out_bf16:
                if k >= 2:
                    outcps[k - 2].wait()

                def pb(r, cc, ep=ep):
                    rv = jnp.full((16,), r, jnp.int32)
                    epc = pconst[ep]
                    for j in range(4):
                        a = plsc.load_gather(ebuf, [epc, rv, eidx[j]])
                        b = plsc.load_gather(ebuf, [epc, rv, oidx[j]])
                        obuf[ep, r, pl.ds(j * 32, 32)] = plsc.pack(
                            a, b, format=plsc.PackFormat.INTERLEAVED)
                    return cc
                lax.fori_loop(0, ESLAB, pb, 0)
                src_slab = obuf.at[ep]
            else:
                src_slab = ebuf.at[ep]
            outcps.append(pltpu.async_copy(
                src_slab, out.at[c, pl.ds(rbase + k * ESLAB, ESLAB)],
                ssem))
        for cp in outcps[-2:]:
            cp.wait()

    return spmm


_spmm_a = _make_spmm(N, False, True)
_spmm_b = _make_spmm(NP, True, False)


def kernel(x, edge_index, edge_weight, W, b, scalar):
    dst = edge_index[0].astype(jnp.int32)
    col = edge_index[1].astype(jnp.int32)
    ew = edge_weight.astype(jnp.float32)
    scal16 = jnp.broadcast_to(scalar.astype(jnp.float32), (16,))

    h1 = _linear(x, W, b)
    o1 = _spmm_a(h1.reshape(2 * N, FH), col, dst, ew, scal16)
    o2 = _spmm_b(o1.reshape(2 * NP, FH), col, dst, ew, scal16)
    return o2[:, :N, :].transpose(1, 0, 2).reshape(N, F)


# speedup vs baseline: 1.0457x; 1.0457x over previous
import functools

import jax
import jax.numpy as jnp
from jax import lax
from jax.experimental import pallas as pl
from jax.experimental.pallas import tpu as pltpu
from jax.experimental.pallas import tpu_sc as plsc

N = 10000
E = 160000
F = 256
FH = 128
EPT = E // 16
CHUNK = 80
NCH = EPT // CHUNK
NP = 10240
NPT = NP // 16
ESLAB = 80
MROWS = 1000


def _mm_body(x_ref, w_ref, b_ref, o_ref):
    o_ref[0] = lax.dot_general(
        x_ref[...], w_ref[...], (((1,), (1,)), ((), ())),
        preferred_element_type=jnp.float32) + b_ref[pl.ds(pl.program_id(0), 1)]


def _linear(x, W, b):
    return pl.pallas_call(
        _mm_body,
        grid=(2, N // MROWS),
        in_specs=[
            pl.BlockSpec((MROWS, F), lambda c, i: (i, 0)),
            pl.BlockSpec((FH, F), lambda c, i: (c, 0)),
            pl.BlockSpec((2, FH), lambda c, i: (0, 0)),
        ],
        out_specs=pl.BlockSpec((1, MROWS, FH), lambda c, i: (c, i, 0)),
        out_shape=jax.ShapeDtypeStruct((2, N, FH), jnp.float32),
    )(x, W, b.reshape(2, FH))


def _make_spmm(table_rows_per_core: int, scale_w: bool):
    TN = table_rows_per_core
    mesh = plsc.VectorSubcoreMesh(core_axis_name="c", subcore_axis_name="s")

    @functools.partial(
        pl.kernel, mesh=mesh,
        out_type=jax.ShapeDtypeStruct((2, NP, FH), jnp.float32),
        compiler_params=pltpu.CompilerParams(
            use_tc_tiling_on_sc=False, needs_layout_passes=False),
        scratch_types=[
            pltpu.VMEM_SHARED((NP, FH), jnp.float32),
            pltpu.VMEM((2, CHUNK, FH), jnp.float32),
            pltpu.VMEM((2, ESLAB, FH), jnp.float32),
            pltpu.VMEM((2, CHUNK), jnp.int32),
            pltpu.VMEM((4, CHUNK), jnp.int32),
            pltpu.VMEM((4, CHUNK), jnp.int32),
            pltpu.VMEM((4, CHUNK), jnp.float32),
            pltpu.VMEM((16,), jnp.float32),
            pltpu.SemaphoreType.DMA,
            pltpu.SemaphoreType.DMA,
            pltpu.SemaphoreType.DMA,
        ],
    )
    def spmm(table, col, dst, ew, scal, out,
             acc, rowsv, ebuf, idxv, colv, dstv, wv, scalv, sem, msem, ssem):
        c = lax.axis_index("c")
        s = lax.axis_index("s")
        zero16 = jnp.zeros((16,), jnp.float32)
        pltpu.sync_copy(scal, scalv)
        sv = scalv[pl.ds(0, 16)]

        def zb(e, cc):
            for f in range(FH // 16):
                rowsv[0, e, pl.ds(f * 16, 16)] = zero16
            return cc
        lax.fori_loop(0, CHUNK, zb, 0)
        for j in range(NPT // CHUNK):
            pltpu.sync_copy(rowsv.at[0],
                            acc.at[pl.ds(s * NPT + j * CHUNK, CHUNK)])
        plsc.subcore_barrier()

        cN = c * TN
        cNv = jnp.full((16,), cN, jnp.int32)
        ebase = s * EPT

        def stage_meta(e0, p):
            cp1 = pltpu.async_copy(col.at[pl.ds(e0, CHUNK)], colv.at[p], msem)
            cp2 = pltpu.async_copy(dst.at[pl.ds(e0, CHUNK)], dstv.at[p], msem)
            cp3 = pltpu.async_copy(ew.at[pl.ds(e0, CHUNK)], wv.at[p], msem)
            return cp1, cp2, cp3

        def build_idx(pi, q):
            def ib(i, c2):
                idxv[pi, pl.ds(i * 16, 16)] = colv[q, pl.ds(i * 16, 16)] + cNv
                return c2
            lax.fori_loop(0, CHUNK // 16, ib, 0)
            if scale_w:
                def wb(i, c2):
                    wv[q, pl.ds(i * 16, 16)] = wv[q, pl.ds(i * 16, 16)] * sv
                    return c2
                lax.fori_loop(0, CHUNK // 16, wb, 0)

        def fire_gather(p):
            return pltpu.async_copy(table.at[idxv.at[p]], rowsv.at[p], sem)

        def wait_gather(p):
            pltpu.make_async_copy(table.at[idxv.at[p]], rowsv.at[p], sem).wait()

        pconst = [jnp.full((16,), pp, jnp.int32) for pp in range(4)]

        def scale(p, q):
            @plsc.parallel_loop(0, CHUNK, 2, unroll=2)
            def sb(e):
                for k in range(2):
                    ws = plsc.load_gather(
                        wv, [pconst[q], jnp.full((16,), e + k, jnp.int32)])
                    for f in range(FH // 16):
                        v = rowsv[p, e + k, pl.ds(f * 16, 16)]
                        rowsv[p, e + k, pl.ds(f * 16, 16)] = v * ws

        def fire_scatter(p, q):
            return pltpu.async_copy(rowsv.at[p], acc.at[dstv.at[q]],
                                    ssem, add=True)

        def wait_scatter(p, q):
            pltpu.make_async_copy(rowsv.at[p], acc.at[dstv.at[q]],
                                  ssem).wait()

        emax = E - CHUNK

        def wait_meta(q):
            for r in (colv, dstv):
                pltpu.make_async_copy(col.at[pl.ds(0, CHUNK)],
                                      r.at[q], msem).wait()
            pltpu.make_async_copy(ew.at[pl.ds(0, CHUNK)],
                                  wv.at[q], msem).wait()

        for cp in stage_meta(ebase, 0):
            cp.wait()
        build_idx(0, 0)
        fire_gather(0)
        stage_meta(ebase + CHUNK, 1)

        def half(i, g, j):
            p = j % 2
            wait_meta((j + 1) % 4)
            build_idx(1 - p, (j + 1) % 4)
            wait_gather(p)
            if j == 0:
                @pl.when(i > 0)
                def _():
                    wait_scatter(1, 3)
            else:
                wait_scatter(1 - p, j - 1)
            fire_gather(1 - p)
            scale(p, j)
            fire_scatter(p, j)
            stage_meta(jnp.minimum(ebase + (g + 2) * CHUNK, emax),
                       (j + 2) % 4)

        def quad_body(i, cc):
            g = 4 * i
            for j in range(4):
                half(i, g + j, j)
            return cc
        lax.fori_loop(0, (NCH - 1) // 4, quad_body, 0)

        wait_meta(1)
        wait_gather(0)
        wait_scatter(1, 3)
        scale(0, 0)
        pltpu.sync_copy(rowsv.at[0], acc.at[dstv.at[0]], add=True)

        plsc.subcore_barrier()
        nslab = NPT // ESLAB
        rbase = s * NPT
        pltpu.async_copy(acc.at[pl.ds(rbase, ESLAB)], ebuf.at[0], msem)
        outcps = []
        for k in range(nslab):
            ep = k % 2
            pltpu.make_async_copy(acc.at[pl.ds(rbase, ESLAB)],
                                  ebuf.at[ep], msem).wait()
            if k + 1 < nslab:
                if k + 1 >= 2:
                    outcps[k - 1].wait()
                pltpu.async_copy(
                    acc.at[pl.ds(rbase + (k + 1) * ESLAB, ESLAB)],
                    ebuf.at[1 - ep], msem)

            def eb(r, cc, ep=ep):
                for f in range(FH // 16):
                    v = ebuf[ep, r, pl.ds(f * 16, 16)]
                    ebuf[ep, r, pl.ds(f * 16, 16)] = jnp.where(
                        v > 0, v, jnp.exp(v) - 1.0)
                return cc
            lax.fori_loop(0, ESLAB, eb, 0)
            outcps.append(pltpu.async_copy(
                ebuf.at[ep], out.at[c, pl.ds(rbase + k * ESLAB, ESLAB)],
                ssem))
        for cp in outcps[-2:]:
            cp.wait()

    return spmm


_spmm_a = _make_spmm(N, False)
_spmm_b = _make_spmm(NP, True)


def kernel(x, edge_index, edge_weight, W, b, scalar):
    dst = edge_index[0].astype(jnp.int32)
    col = edge_index[1].astype(jnp.int32)
    ew = edge_weight.astype(jnp.float32)
    scal16 = jnp.broadcast_to(scalar.astype(jnp.float32), (16,))

    h1 = _linear(x, W, b)
    o1 = _spmm_a(h1.reshape(2 * N, FH), col, dst, ew, scal16)
    o2 = _spmm_b(o1.reshape(2 * NP, FH), col, dst, ew, scal16)
    return o2[:, :N, :].transpose(1, 0, 2).reshape(N, F)


# speedup vs baseline: 1.0583x; 1.0120x over previous
import functools

import jax
import jax.numpy as jnp
from jax import lax
from jax.experimental import pallas as pl
from jax.experimental.pallas import tpu as pltpu
from jax.experimental.pallas import tpu_sc as plsc

N = 10000
E = 160000
F = 256
FH = 128
EPT = E // 16
CHUNK = 80
NCH = EPT // CHUNK
NP = 10240
NPT = NP // 16
ESLAB = 80
MROWS = 1000


def _mm_body(x_ref, w_ref, b_ref, o_ref):
    o_ref[0] = lax.dot_general(
        x_ref[...], w_ref[...], (((1,), (1,)), ((), ())),
        preferred_element_type=jnp.float32) + b_ref[pl.ds(pl.program_id(0), 1)]


def _linear(x, W, b):
    return pl.pallas_call(
        _mm_body,
        grid=(2, N // MROWS),
        in_specs=[
            pl.BlockSpec((MROWS, F), lambda c, i: (i, 0)),
            pl.BlockSpec((FH, F), lambda c, i: (c, 0)),
            pl.BlockSpec((2, FH), lambda c, i: (0, 0)),
        ],
        out_specs=pl.BlockSpec((1, MROWS, FH), lambda c, i: (c, i, 0)),
        out_shape=jax.ShapeDtypeStruct((2, N, FH), jnp.float32),
    )(x, W, b.reshape(2, FH))


def _make_gcn():
    mesh = plsc.VectorSubcoreMesh(core_axis_name="c", subcore_axis_name="s")

    @functools.partial(
        pl.kernel, mesh=mesh,
        out_type=jax.ShapeDtypeStruct((2, NP, FH), jnp.float32),
        compiler_params=pltpu.CompilerParams(
            use_tc_tiling_on_sc=False, needs_layout_passes=False),
        scratch_types=[
            pltpu.HBM((2 * NP, FH), jnp.float32),
            pltpu.VMEM_SHARED((NP, FH), jnp.float32),
            pltpu.VMEM((2, CHUNK, FH), jnp.float32),
            pltpu.VMEM((2, ESLAB, FH), jnp.float32),
            pltpu.VMEM((2, CHUNK), jnp.int32),
            pltpu.VMEM((4, CHUNK), jnp.int32),
            pltpu.VMEM((4, CHUNK), jnp.int32),
            pltpu.VMEM((4, CHUNK), jnp.float32),
            pltpu.VMEM((16,), jnp.float32),
            pltpu.SemaphoreType.DMA,
            pltpu.SemaphoreType.DMA,
            pltpu.SemaphoreType.DMA,
        ],
    )
    def gcn(table, col, dst, ew, scal, out,
            hact, acc, rowsv, ebuf, idxv, colv, dstv, wv, scalv,
            sem, msem, ssem):
        c = lax.axis_index("c")
        s = lax.axis_index("s")
        zero16 = jnp.zeros((16,), jnp.float32)
        iota16 = lax.iota(jnp.int32, 16)
        pltpu.sync_copy(scal, scalv)
        sv = scalv[pl.ds(0, 16)]
        ebase = s * EPT
        emax = E - CHUNK
        pconst = [jnp.full((16,), pp, jnp.int32) for pp in range(4)]

        def run_phase(tbl, TN, scale_w, out_ref):
            cNv = jnp.full((16,), c * TN, jnp.int32)

            def zb(e, cc):
                for f in range(FH // 16):
                    rowsv[0, e, pl.ds(f * 16, 16)] = zero16
                return cc
            lax.fori_loop(0, CHUNK, zb, 0)
            for j in range(NPT // CHUNK):
                pltpu.sync_copy(rowsv.at[0],
                                acc.at[pl.ds(s * NPT + j * CHUNK, CHUNK)])
            plsc.subcore_barrier()

            def stage_meta(e0, q):
                cp1 = pltpu.async_copy(col.at[pl.ds(e0, CHUNK)],
                                       colv.at[q], msem)
                cp2 = pltpu.async_copy(dst.at[pl.ds(e0, CHUNK)],
                                       dstv.at[q], msem)
                cp3 = pltpu.async_copy(ew.at[pl.ds(e0, CHUNK)],
                                       wv.at[q], msem)
                return cp1, cp2, cp3

            def wait_meta(q):
                for r in (colv, dstv):
                    pltpu.make_async_copy(col.at[pl.ds(0, CHUNK)],
                                          r.at[q], msem).wait()
                pltpu.make_async_copy(ew.at[pl.ds(0, CHUNK)],
                                      wv.at[q], msem).wait()

            def build_idx(pi, q):
                def ib(i, c_):
                    idxv[pi, pl.ds(i * 16, 16)] = (
                        colv[q, pl.ds(i * 16, 16)] + cNv)
                    return c_
                lax.fori_loop(0, CHUNK // 16, ib, 0)
                if scale_w:
                    def wb(i, c_):
                        wv[q, pl.ds(i * 16, 16)] = (
                            wv[q, pl.ds(i * 16, 16)] * sv)
                        return c_
                    lax.fori_loop(0, CHUNK // 16, wb, 0)

            def fire_gather(p):
                return pltpu.async_copy(tbl.at[idxv.at[p]], rowsv.at[p], sem)

            def wait_gather(p):
                pltpu.make_async_copy(tbl.at[idxv.at[p]], rowsv.at[p],
                                      sem).wait()

            def scale(p, q):
                @plsc.parallel_loop(0, CHUNK, 2, unroll=2)
                def sb(e):
                    for k in range(2):
                        ws = plsc.load_gather(
                            wv, [pconst[q], jnp.full((16,), e + k, jnp.int32)])
                        for f in range(FH // 16):
                            v = rowsv[p, e + k, pl.ds(f * 16, 16)]
                            rowsv[p, e + k, pl.ds(f * 16, 16)] = v * ws

            def fire_scatter(p, q):
                return pltpu.async_copy(rowsv.at[p], acc.at[dstv.at[q]],
                                        ssem, add=True)

            def wait_scatter(p, q):
                pltpu.make_async_copy(rowsv.at[p], acc.at[dstv.at[q]],
                                      ssem).wait()

            for cp in stage_meta(ebase, 0):
                cp.wait()
            build_idx(0, 0)
            fire_gather(0)
            stage_meta(ebase + CHUNK, 1)

            def half(i, g, j):
                p = j % 2
                wait_meta((j + 1) % 4)
                build_idx(1 - p, (j + 1) % 4)
                wait_gather(p)
                if j == 0:
                    @pl.when(i > 0)
                    def _():
                        wait_scatter(1, 3)
                else:
                    wait_scatter(1 - p, j - 1)
                fire_gather(1 - p)
                scale(p, j)
                fire_scatter(p, j)
                stage_meta(jnp.minimum(ebase + (g + 2) * CHUNK, emax),
                           (j + 2) % 4)

            def quad_body(i, cc):
                g = 4 * i
                for j in range(4):
                    half(i, g + j, j)
                return cc
            lax.fori_loop(0, (NCH - 1) // 4, quad_body, 0)

            wait_meta(1)
            wait_gather(0)
            wait_scatter(1, 3)
            scale(0, 0)
            pltpu.sync_copy(rowsv.at[0], acc.at[dstv.at[0]], add=True)

            plsc.subcore_barrier()
            nslab = NPT // ESLAB
            rbase = s * NPT
            pltpu.async_copy(acc.at[pl.ds(rbase, ESLAB)], ebuf.at[0], msem)
            outcps = []
            for k in range(nslab):
                ep = k % 2
                pltpu.make_async_copy(acc.at[pl.ds(rbase, ESLAB)],
                                      ebuf.at[ep], msem).wait()
                if k + 1 < nslab:
                    if k + 1 >= 2:
                        outcps[k - 1].wait()
                    pltpu.async_copy(
                        acc.at[pl.ds(rbase + (k + 1) * ESLAB, ESLAB)],
                        ebuf.at[1 - ep], msem)

                def eb(r, cc, ep=ep):
                    for f in range(FH // 16):
                        v = ebuf[ep, r, pl.ds(f * 16, 16)]
                        ebuf[ep, r, pl.ds(f * 16, 16)] = jnp.where(
                            v > 0, v, jnp.exp(v) - 1.0)
                    return cc
                lax.fori_loop(0, ESLAB, eb, 0)
                outcps.append(pltpu.async_copy(
                    ebuf.at[ep], out_ref.at[pl.ds(rbase + k * ESLAB, ESLAB)],
                    ssem))
            for cp in outcps[-2:]:
                cp.wait()
            plsc.subcore_barrier()

        hview = hact.at[pl.ds(c * NP, NP)]
        run_phase(table, N, False, hview)
        run_phase(hact, NP, True, out.at[c])

    return gcn


_gcn = _make_gcn()


def kernel(x, edge_index, edge_weight, W, b, scalar):
    dst = edge_index[0].astype(jnp.int32)
    col = edge_index[1].astype(jnp.int32)
    ew = edge_weight.astype(jnp.float32)
    scal16 = jnp.broadcast_to(scalar.astype(jnp.float32), (16,))

    h1 = _linear(x, W, b)
    o2 = _gcn(h1.reshape(2 * N, FH), col, dst, ew, scal16)
    return o2[:, :N, :].transpose(1, 0, 2).reshape(N, F)
